# Initial kernel scaffold; baseline (speedup 1.0000x reference)
import jax
import jax.numpy as jnp
from jax.experimental import pallas as pl


def kernel(x, edge_index_rel0, edge_index_rel1, W1_rel0, W2_rel0, g1_rel0, b1_rel0, g2_rel0, b2_rel0, W1_rel1, W2_rel1, g1_rel1, b1_rel1, g2_rel1, b2_rel1):
    raise NotImplementedError("write your pallas kernel here")


# speedup vs baseline: 4.5139x; 4.5139x over previous
import functools

import jax
import jax.numpy as jnp
from jax import lax
from jax.experimental import pallas as pl
from jax.experimental.pallas import tpu as pltpu
from jax.experimental.pallas import tpu_sc as plsc

_N = 10000
_NP = 10240
_D = 128
_DH = 64
_E = 320000
_EPS = 1e-5

_CH = 100
_NCHUNK = _E // _CH
_NSUB = 16
_CPT = _NCHUNK // _NSUB
_RPT = _NP // _NSUB
_RCH = 128


def _sc_body(xlo_hbm, xhi_hbm, srcs_hbm, dsts_hbm, outlo_hbm, outhi_hbm,
             src_all, dst_all, rows, stage, agg, sem):
  c = lax.axis_index("c")
  s = lax.axis_index("s")
  row0 = s * _RPT

  pltpu.sync_copy(srcs_hbm.at[c, pl.ds(s * _CPT, _CPT)], src_all)
  pltpu.sync_copy(dsts_hbm.at[c, pl.ds(s * _CPT, _CPT)], dst_all)

  for x_hbm, out_hbm in ((xlo_hbm, outlo_hbm), (xhi_hbm, outhi_hbm)):
    for i in range(_RPT // _RCH):
      pltpu.sync_copy(x_hbm.at[pl.ds(row0 + i * _RCH, _RCH)], stage)
      pltpu.sync_copy(stage, agg.at[pl.ds(row0 + i * _RCH, _RCH)])
    plsc.subcore_barrier()

    def step(j, carry):
      pltpu.async_copy(x_hbm.at[src_all.at[j]], rows, sem).wait()
      pltpu.sync_copy(rows, agg.at[dst_all.at[j]], add=True)
      return carry

    lax.fori_loop(0, _CPT, step, 0)
    plsc.subcore_barrier()

    for i in range(_RPT // _RCH):
      pltpu.sync_copy(agg.at[pl.ds(row0 + i * _RCH, _RCH)], stage)
      pltpu.sync_copy(
          stage, out_hbm.at[pl.ds(c * _NP + row0 + i * _RCH, _RCH)])


_sc_aggregate = functools.partial(
    pl.kernel,
    mesh=plsc.VectorSubcoreMesh(core_axis_name="c", subcore_axis_name="s"),
    compiler_params=pltpu.CompilerParams(use_tc_tiling_on_sc=False),
    out_type=(jax.ShapeDtypeStruct((2 * _NP, _DH), jnp.float32),
              jax.ShapeDtypeStruct((2 * _NP, _DH), jnp.float32)),
    scratch_types=[
        pltpu.VMEM((_CPT, _CH), jnp.int32),
        pltpu.VMEM((_CPT, _CH), jnp.int32),
        pltpu.VMEM((_CH, _DH), jnp.float32),
        pltpu.VMEM((_RCH, _DH), jnp.float32),
        pltpu.VMEM_SHARED((_NP, _DH), jnp.float32),
        pltpu.SemaphoreType.DMA,
    ],
)(_sc_body)


def _bn_relu(t, g, b):
  mu = jnp.mean(t, axis=0, keepdims=True)
  var = jnp.mean((t - mu) * (t - mu), axis=0, keepdims=True)
  return jnp.maximum((t - mu) * lax.rsqrt(var + _EPS) * g + b, 0.0)


def _mlp(hlo, hhi, w1tlo, w1thi, w2t, g1, b1, g2, b2):
  t = (jnp.dot(hlo, w1tlo, preferred_element_type=jnp.float32) +
       jnp.dot(hhi, w1thi, preferred_element_type=jnp.float32))
  t = _bn_relu(t, g1, b1)
  t = jnp.dot(t, w2t, preferred_element_type=jnp.float32)
  return _bn_relu(t, g2, b2)


def _tc_body(h2lo, h2hi, w1tlo0, w1thi0, w2t0, w1tlo1, w1thi1, w2t1,
             g10, b10, g20, b20, g11, b11, g21, b21, out):
  o0 = _mlp(h2lo[0:_N, :], h2hi[0:_N, :],
            w1tlo0[...], w1thi0[...], w2t0[...],
            g10[...], b10[...], g20[...], b20[...])
  o1 = _mlp(h2lo[_NP:_NP + _N, :], h2hi[_NP:_NP + _N, :],
            w1tlo1[...], w1thi1[...], w2t1[...],
            g11[...], b11[...], g21[...], b21[...])
  out[...] = o0 + o1


_tc_mlp = pl.pallas_call(
    _tc_body,
    out_shape=jax.ShapeDtypeStruct((_N, _D), jnp.float32),
)


@jax.jit
def kernel(x, edge_index_rel0, edge_index_rel1,
           W1_rel0, W2_rel0, g1_rel0, b1_rel0, g2_rel0, b2_rel0,
           W1_rel1, W2_rel1, g1_rel1, b1_rel1, g2_rel1, b2_rel1):
  xp = jnp.pad(x, ((0, _NP - _N), (0, 0)))
  srcs = jnp.stack([edge_index_rel0[0].reshape(_NCHUNK, _CH),
                    edge_index_rel1[0].reshape(_NCHUNK, _CH)])
  dsts = jnp.stack([edge_index_rel0[1].reshape(_NCHUNK, _CH),
                    edge_index_rel1[1].reshape(_NCHUNK, _CH)])
  h2lo, h2hi = _sc_aggregate(xp[:, :_DH], xp[:, _DH:], srcs, dsts)
  w1t0 = W1_rel0.T
  w1t1 = W1_rel1.T
  vec = lambda v: v.reshape(1, _D)
  return _tc_mlp(h2lo, h2hi,
                 w1t0[:_DH], w1t0[_DH:], W2_rel0.T,
                 w1t1[:_DH], w1t1[_DH:], W2_rel1.T,
                 vec(g1_rel0), vec(b1_rel0), vec(g2_rel0), vec(b2_rel0),
                 vec(g1_rel1), vec(b1_rel1), vec(g2_rel1), vec(b2_rel1))


# speedup vs baseline: 8.0737x; 1.7887x over previous
import functools

import jax
import jax.numpy as jnp
from jax import lax
from jax.experimental import pallas as pl
from jax.experimental.pallas import tpu as pltpu
from jax.experimental.pallas import tpu_sc as plsc

_N = 10000
_NP = 10240
_D = 128
_DH = 64
_E = 320000
_EPS = 1e-5

_CH = 100
_NCHUNK = _E // _CH
_NSUB = 16
_CPT = _NCHUNK // _NSUB
_RPT = _NP // _NSUB
_RCH = 128


_NBUF = 4
_NSTEP = _CPT // _NBUF


def _sc_body(xlo_hbm, xhi_hbm, srcs_hbm, dsts_hbm, outlo_hbm, outhi_hbm,
             src_all, dst_all, bufs, stage, agg, gsems, ssems):
  c = lax.axis_index("c")
  s = lax.axis_index("s")
  row0 = s * _RPT

  pltpu.sync_copy(srcs_hbm.at[c, pl.ds(s * _CPT, _CPT)], src_all)
  pltpu.sync_copy(dsts_hbm.at[c, pl.ds(s * _CPT, _CPT)], dst_all)

  for x_hbm, out_hbm in ((xlo_hbm, outlo_hbm), (xhi_hbm, outhi_hbm)):
    for i in range(_RPT // _RCH):
      pltpu.sync_copy(x_hbm.at[pl.ds(row0 + i * _RCH, _RCH)], stage)
      pltpu.sync_copy(stage, agg.at[pl.ds(row0 + i * _RCH, _RCH)])
    plsc.subcore_barrier()

    for b in range(_NBUF):
      pltpu.async_copy(x_hbm.at[src_all.at[b]], bufs.at[b], gsems[b])

    def step(i, carry):
      j0 = i * _NBUF
      for b in range(_NBUF):
        pltpu.make_async_copy(
            x_hbm.at[src_all.at[j0 + b]], bufs.at[b], gsems[b]).wait()
        pltpu.async_copy(
            bufs.at[b], agg.at[dst_all.at[j0 + b]], ssems[b], add=True)
      for b in range(_NBUF):
        pltpu.make_async_copy(
            bufs.at[b], agg.at[dst_all.at[j0 + b]], ssems[b]).wait()

        @pl.when(j0 + b + _NBUF < _CPT)
        def _():
          pltpu.async_copy(
              x_hbm.at[src_all.at[j0 + b + _NBUF]], bufs.at[b], gsems[b])
      return carry

    lax.fori_loop(0, _NSTEP, step, 0)
    plsc.subcore_barrier()

    for i in range(_RPT // _RCH):
      pltpu.sync_copy(agg.at[pl.ds(row0 + i * _RCH, _RCH)], stage)
      pltpu.sync_copy(
          stage, out_hbm.at[pl.ds(c * _NP + row0 + i * _RCH, _RCH)])


_sc_aggregate = functools.partial(
    pl.kernel,
    mesh=plsc.VectorSubcoreMesh(core_axis_name="c", subcore_axis_name="s"),
    compiler_params=pltpu.CompilerParams(use_tc_tiling_on_sc=False),
    out_type=(jax.ShapeDtypeStruct((2 * _NP, _DH), jnp.float32),
              jax.ShapeDtypeStruct((2 * _NP, _DH), jnp.float32)),
    scratch_types=[
        pltpu.VMEM((_CPT, _CH), jnp.int32),
        pltpu.VMEM((_CPT, _CH), jnp.int32),
        pltpu.VMEM((_NBUF, _CH, _DH), jnp.float32),
        pltpu.VMEM((_RCH, _DH), jnp.float32),
        pltpu.VMEM_SHARED((_NP, _DH), jnp.float32),
        [pltpu.SemaphoreType.DMA] * _NBUF,
        [pltpu.SemaphoreType.DMA] * _NBUF,
    ],
)(_sc_body)


def _bn_relu(t, g, b):
  mu = jnp.mean(t, axis=0, keepdims=True)
  var = jnp.mean((t - mu) * (t - mu), axis=0, keepdims=True)
  return jnp.maximum((t - mu) * lax.rsqrt(var + _EPS) * g + b, 0.0)


def _mlp(hlo, hhi, w1tlo, w1thi, w2t, g1, b1, g2, b2):
  t = (jnp.dot(hlo, w1tlo, preferred_element_type=jnp.float32) +
       jnp.dot(hhi, w1thi, preferred_element_type=jnp.float32))
  t = _bn_relu(t, g1, b1)
  t = jnp.dot(t, w2t, preferred_element_type=jnp.float32)
  return _bn_relu(t, g2, b2)


def _tc_body(h2lo, h2hi, w1tlo0, w1thi0, w2t0, w1tlo1, w1thi1, w2t1,
             g10, b10, g20, b20, g11, b11, g21, b21, out):
  o0 = _mlp(h2lo[0:_N, :], h2hi[0:_N, :],
            w1tlo0[...], w1thi0[...], w2t0[...],
            g10[...], b10[...], g20[...], b20[...])
  o1 = _mlp(h2lo[_NP:_NP + _N, :], h2hi[_NP:_NP + _N, :],
            w1tlo1[...], w1thi1[...], w2t1[...],
            g11[...], b11[...], g21[...], b21[...])
  out[...] = o0 + o1


_tc_mlp = pl.pallas_call(
    _tc_body,
    out_shape=jax.ShapeDtypeStruct((_N, _D), jnp.float32),
)


@jax.jit
def kernel(x, edge_index_rel0, edge_index_rel1,
           W1_rel0, W2_rel0, g1_rel0, b1_rel0, g2_rel0, b2_rel0,
           W1_rel1, W2_rel1, g1_rel1, b1_rel1, g2_rel1, b2_rel1):
  xp = jnp.pad(x, ((0, _NP - _N), (0, 0)))
  srcs = jnp.stack([edge_index_rel0[0].reshape(_NCHUNK, _CH),
                    edge_index_rel1[0].reshape(_NCHUNK, _CH)])
  dsts = jnp.stack([edge_index_rel0[1].reshape(_NCHUNK, _CH),
                    edge_index_rel1[1].reshape(_NCHUNK, _CH)])
  h2lo, h2hi = _sc_aggregate(xp[:, :_DH], xp[:, _DH:], srcs, dsts)
  w1t0 = W1_rel0.T
  w1t1 = W1_rel1.T
  vec = lambda v: v.reshape(1, _D)
  return _tc_mlp(h2lo, h2hi,
                 w1t0[:_DH], w1t0[_DH:], W2_rel0.T,
                 w1t1[:_DH], w1t1[_DH:], W2_rel1.T,
                 vec(g1_rel0), vec(b1_rel0), vec(g2_rel0), vec(b2_rel0),
                 vec(g1_rel1), vec(b1_rel1), vec(g2_rel1), vec(b2_rel1))


# speedup vs baseline: 8.8272x; 1.0933x over previous
import functools

import jax
import jax.numpy as jnp
from jax import lax
from jax.experimental import pallas as pl
from jax.experimental.pallas import tpu as pltpu
from jax.experimental.pallas import tpu_sc as plsc

_N = 10000
_NP = 10240
_D = 128
_DH = 64
_E = 320000
_EPS = 1e-5

_CH = 128
_EP = 327680
_NCHUNK = _EP // _CH
_NSUB = 16
_CPT = _NCHUNK // _NSUB
_RPT = _NP // _NSUB
_RCH = 128
_NBUF = 4
_NSTEP = _CPT // _NBUF


def _sc_body(xlo_hbm, xhi_hbm, srcs_hbm, dsts_hbm,
             outlo_hbm, outhi_hbm, src_all, dst_all, bufs, agg,
             gsems, ssems):
  c = lax.axis_index("c")
  s = lax.axis_index("s")
  row0 = s * _RPT

  pltpu.sync_copy(srcs_hbm.at[c, pl.ds(s * _CPT, _CPT)], src_all)
  pltpu.sync_copy(dsts_hbm.at[c, pl.ds(s * _CPT, _CPT)], dst_all)

  for x_hbm, out_hbm in ((xlo_hbm, outlo_hbm), (xhi_hbm, outhi_hbm)):
    for i in range(_RPT // _RCH):
      pltpu.sync_copy(x_hbm.at[pl.ds(row0 + i * _RCH, _RCH)], bufs.at[0])
      pltpu.sync_copy(bufs.at[0], agg.at[pl.ds(row0 + i * _RCH, _RCH)])
    plsc.subcore_barrier()

    for b in range(_NBUF):
      pltpu.async_copy(x_hbm.at[src_all.at[b]], bufs.at[b], gsems[b])

    def step(i, carry):
      j0 = i * _NBUF
      for b in range(_NBUF):
        pltpu.make_async_copy(
            x_hbm.at[src_all.at[j0 + b]], bufs.at[b], gsems[b]).wait()
        pltpu.async_copy(
            bufs.at[b], agg.at[dst_all.at[j0 + b]], ssems[b], add=True)
      for b in range(_NBUF):
        pltpu.make_async_copy(
            bufs.at[b], agg.at[dst_all.at[j0 + b]], ssems[b]).wait()

        @pl.when(j0 + b + _NBUF < _CPT)
        def _():
          pltpu.async_copy(
              x_hbm.at[src_all.at[j0 + b + _NBUF]], bufs.at[b], gsems[b])
      return carry

    lax.fori_loop(0, _NSTEP, step, 0)
    plsc.subcore_barrier()

    for i in range(_RPT // _RCH):
      pltpu.sync_copy(agg.at[pl.ds(row0 + i * _RCH, _RCH)], bufs.at[0])
      pltpu.sync_copy(
          bufs.at[0], out_hbm.at[pl.ds(c * _NP + row0 + i * _RCH, _RCH)])


_sc_aggregate = functools.partial(
    pl.kernel,
    mesh=plsc.VectorSubcoreMesh(core_axis_name="c", subcore_axis_name="s"),
    compiler_params=pltpu.CompilerParams(use_tc_tiling_on_sc=False),
    out_type=(jax.ShapeDtypeStruct((2 * _NP, _DH), jnp.float32),
              jax.ShapeDtypeStruct((2 * _NP, _DH), jnp.float32)),
    scratch_types=[
        pltpu.VMEM((_CPT, _CH), jnp.int32),
        pltpu.VMEM((_CPT, _CH), jnp.int32),
        pltpu.VMEM((_NBUF, _CH, _DH), jnp.float32),
        pltpu.VMEM_SHARED((_NP, _DH), jnp.float32),
        [pltpu.SemaphoreType.DMA] * _NBUF,
        [pltpu.SemaphoreType.DMA] * _NBUF,
    ],
)(_sc_body)


def _bn_relu(t, g, b):
  mu = jnp.mean(t, axis=0, keepdims=True)
  var = jnp.mean((t - mu) * (t - mu), axis=0, keepdims=True)
  return jnp.maximum((t - mu) * lax.rsqrt(var + _EPS) * g + b, 0.0)


def _mlp(hlo, hhi, w1tlo, w1thi, w2t, g1, b1, g2, b2):
  t = (jnp.dot(hlo, w1tlo, preferred_element_type=jnp.float32) +
       jnp.dot(hhi, w1thi, preferred_element_type=jnp.float32))
  t = _bn_relu(t, g1, b1)
  t = jnp.dot(t, w2t, preferred_element_type=jnp.float32)
  return _bn_relu(t, g2, b2)


def _tc_body(h2lo, h2hi, w1tlo0, w1thi0, w2t0, w1tlo1, w1thi1, w2t1,
             g10, b10, g20, b20, g11, b11, g21, b21, out):
  o0 = _mlp(h2lo[0:_N, :], h2hi[0:_N, :],
            w1tlo0[...], w1thi0[...], w2t0[...],
            g10[...], b10[...], g20[...], b20[...])
  o1 = _mlp(h2lo[_NP:_NP + _N, :], h2hi[_NP:_NP + _N, :],
            w1tlo1[...], w1thi1[...], w2t1[...],
            g11[...], b11[...], g21[...], b21[...])
  out[...] = o0 + o1


_tc_mlp = pl.pallas_call(
    _tc_body,
    out_shape=jax.ShapeDtypeStruct((_N, _D), jnp.float32),
)


def _pad_edges(idx, fill):
  return jnp.concatenate([idx, fill]).reshape(_NCHUNK, _CH)


@jax.jit
def kernel(x, edge_index_rel0, edge_index_rel1,
           W1_rel0, W2_rel0, g1_rel0, b1_rel0, g2_rel0, b2_rel0,
           W1_rel1, W2_rel1, g1_rel1, b1_rel1, g2_rel1, b2_rel1):
  xp = jnp.pad(x, ((0, _NP - _N), (0, 0)))
  fill = _N + jnp.arange(_EP - _E, dtype=jnp.int32) % (_NP - _N)
  srcs = jnp.stack([_pad_edges(edge_index_rel0[0], fill),
                    _pad_edges(edge_index_rel1[0], fill)])
  dsts = jnp.stack([_pad_edges(edge_index_rel0[1], fill),
                    _pad_edges(edge_index_rel1[1], fill)])
  h2lo, h2hi = _sc_aggregate(xp[:, :_DH], xp[:, _DH:], srcs, dsts)
  w1t0 = W1_rel0.T
  w1t1 = W1_rel1.T
  vec = lambda v: v.reshape(1, _D)
  return _tc_mlp(h2lo, h2hi,
                 w1t0[:_DH], w1t0[_DH:], W2_rel0.T,
                 w1t1[:_DH], w1t1[_DH:], W2_rel1.T,
                 vec(g1_rel0), vec(b1_rel0), vec(g2_rel0), vec(b2_rel0),
                 vec(g1_rel1), vec(b1_rel1), vec(g2_rel1), vec(b2_rel1))


# speedup vs baseline: 10.0351x; 1.1368x over previous
import functools

import jax
import jax.numpy as jnp
from jax import lax
from jax.experimental import pallas as pl
from jax.experimental.pallas import tpu as pltpu
from jax.experimental.pallas import tpu_sc as plsc

_N = 10000
_NP = 10240
_D = 128
_E = 320000
_EPS = 1e-5

_CH = 128
_EP = 327680
_NCHUNK = _EP // _CH
_NSUB = 16
_CPT = _NCHUNK // _NSUB
_RPT = _NP // _NSUB
_RCH = 128
_IB = 16
_NBLK = _CPT // _IB


def _sc_body(x_hbm, srcs_hbm, dsts_hbm, out_hbm,
             sidx, didx, bufs, agg, gsems, ssems, sisems, disems):
  c = lax.axis_index("c")
  s = lax.axis_index("s")
  row0 = s * _RPT
  blk0 = s * _CPT

  def idx_start(k, p):
    pltpu.async_copy(
        srcs_hbm.at[c, pl.ds(blk0 + k * _IB, _IB)], sidx.at[p], sisems[p])
    pltpu.async_copy(
        dsts_hbm.at[c, pl.ds(blk0 + k * _IB, _IB)], didx.at[p], disems[p])

  def idx_wait(k, p):
    pltpu.make_async_copy(
        srcs_hbm.at[c, pl.ds(blk0 + k * _IB, _IB)], sidx.at[p],
        sisems[p]).wait()
    pltpu.make_async_copy(
        dsts_hbm.at[c, pl.ds(blk0 + k * _IB, _IB)], didx.at[p],
        disems[p]).wait()

  idx_start(0, 0)

  for i in range(_RPT // _RCH):
    pltpu.sync_copy(x_hbm.at[pl.ds(row0 + i * _RCH, _RCH)], bufs.at[0])
    pltpu.sync_copy(bufs.at[0], agg.at[pl.ds(row0 + i * _RCH, _RCH)])
  plsc.subcore_barrier()

  def gather_start(p, j, b):
    pltpu.async_copy(x_hbm.at[sidx.at[p, j]], bufs.at[b], gsems[b])

  def gather_wait(p, j, b):
    pltpu.make_async_copy(
        x_hbm.at[sidx.at[p, j]], bufs.at[b], gsems[b]).wait()

  def scatter_start(p, j, b):
    pltpu.async_copy(bufs.at[b], agg.at[didx.at[p, j]], ssems[b], add=True)

  def scatter_wait(p, j, b):
    pltpu.make_async_copy(
        bufs.at[b], agg.at[didx.at[p, j]], ssems[b]).wait()

  def block(k, p):
    @pl.when(k + 1 < _NBLK)
    def _():
      idx_start(k + 1, 1 - p)

    idx_wait(k, p)
    gather_start(p, 0, 0)
    gather_start(p, 1, 1)
    for j in range(_IB):
      b = j % 2
      gather_wait(p, j, b)
      scatter_start(p, j, b)
      scatter_wait(p, j, b)
      if j + 2 < _IB:
        gather_start(p, j + 2, b)

  def two_blocks(i, carry):
    block(2 * i, 0)
    block(2 * i + 1, 1)
    return carry

  lax.fori_loop(0, _NBLK // 2, two_blocks, 0)
  plsc.subcore_barrier()

  for i in range(_RPT // _RCH):
    pltpu.sync_copy(agg.at[pl.ds(row0 + i * _RCH, _RCH)], bufs.at[0])
    pltpu.sync_copy(
        bufs.at[0], out_hbm.at[pl.ds(c * _NP + row0 + i * _RCH, _RCH)])


_sc_aggregate = functools.partial(
    pl.kernel,
    mesh=plsc.VectorSubcoreMesh(core_axis_name="c", subcore_axis_name="s"),
    compiler_params=pltpu.CompilerParams(use_tc_tiling_on_sc=False),
    out_type=jax.ShapeDtypeStruct((2 * _NP, _D), jnp.float32),
    scratch_types=[
        pltpu.VMEM((2, _IB, _CH), jnp.int32),
        pltpu.VMEM((2, _IB, _CH), jnp.int32),
        pltpu.VMEM((2, _CH, _D), jnp.float32),
        pltpu.VMEM_SHARED((_NP, _D), jnp.float32),
        [pltpu.SemaphoreType.DMA] * 2,
        [pltpu.SemaphoreType.DMA] * 2,
        [pltpu.SemaphoreType.DMA] * 2,
        [pltpu.SemaphoreType.DMA] * 2,
    ],
)(_sc_body)


def _bn_relu(t, g, b):
  mu = jnp.mean(t, axis=0, keepdims=True)
  var = jnp.mean((t - mu) * (t - mu), axis=0, keepdims=True)
  return jnp.maximum((t - mu) * lax.rsqrt(var + _EPS) * g + b, 0.0)


def _mlp(h, w1t, w2t, g1, b1, g2, b2):
  t = jnp.dot(h, w1t, preferred_element_type=jnp.float32)
  t = _bn_relu(t, g1, b1)
  t = jnp.dot(t, w2t, preferred_element_type=jnp.float32)
  return _bn_relu(t, g2, b2)


def _tc_body(h2, w1t0, w2t0, w1t1, w2t1,
             g10, b10, g20, b20, g11, b11, g21, b21, out):
  o0 = _mlp(h2[0:_N, :], w1t0[...], w2t0[...],
            g10[...], b10[...], g20[...], b20[...])
  o1 = _mlp(h2[_NP:_NP + _N, :], w1t1[...], w2t1[...],
            g11[...], b11[...], g21[...], b21[...])
  out[...] = o0 + o1


_tc_mlp = pl.pallas_call(
    _tc_body,
    out_shape=jax.ShapeDtypeStruct((_N, _D), jnp.float32),
)


def _pad_edges(idx, fill):
  return jnp.concatenate([idx, fill]).reshape(_NCHUNK, _CH)


@jax.jit
def kernel(x, edge_index_rel0, edge_index_rel1,
           W1_rel0, W2_rel0, g1_rel0, b1_rel0, g2_rel0, b2_rel0,
           W1_rel1, W2_rel1, g1_rel1, b1_rel1, g2_rel1, b2_rel1):
  xp = jnp.pad(x, ((0, _NP - _N), (0, 0)))
  fill = _N + jnp.arange(_EP - _E, dtype=jnp.int32) % (_NP - _N)
  srcs = jnp.stack([_pad_edges(edge_index_rel0[0], fill),
                    _pad_edges(edge_index_rel1[0], fill)])
  dsts = jnp.stack([_pad_edges(edge_index_rel0[1], fill),
                    _pad_edges(edge_index_rel1[1], fill)])
  h2 = _sc_aggregate(xp, srcs, dsts)
  vec = lambda v: v.reshape(1, _D)
  return _tc_mlp(h2, W1_rel0.T, W2_rel0.T, W1_rel1.T, W2_rel1.T,
                 vec(g1_rel0), vec(b1_rel0), vec(g2_rel0), vec(b2_rel0),
                 vec(g1_rel1), vec(b1_rel1), vec(g2_rel1), vec(b2_rel1))


# speedup vs baseline: 10.8119x; 1.0774x over previous
import functools

import jax
import jax.numpy as jnp
from jax import lax
from jax.experimental import pallas as pl
from jax.experimental.pallas import tpu as pltpu
from jax.experimental.pallas import tpu_sc as plsc

_N = 10000
_D = 128
_E = 320000
_EPS = 1e-5

_CH = 128
_NCHUNK = _E // _CH
_NSUB = 16
_CPT = 156
_NTAIL = _NCHUNK - _NSUB * _CPT
_RPT = _N // _NSUB
_RCH = 125
_IB = 12
_NBLK = _CPT // _IB


def _sc_body(x_hbm, e0_hbm, e1_hbm, out_hbm,
             sidx, didx, bufs, agg, gsems, ssems, sisems, disems):
  c = lax.axis_index("c")
  s = lax.axis_index("s")
  row0 = s * _RPT
  blk0 = s * _CPT

  def gather_start(p, j, b):
    pltpu.async_copy(x_hbm.at[sidx.at[p, j]], bufs.at[b], gsems[b])

  def gather_wait(p, j, b):
    pltpu.make_async_copy(
        x_hbm.at[sidx.at[p, j]], bufs.at[b], gsems[b]).wait()

  def scatter_start(p, j, b):
    pltpu.async_copy(bufs.at[b], agg.at[didx.at[p, j]], ssems[b], add=True)

  def scatter_wait(p, j, b):
    pltpu.make_async_copy(
        bufs.at[b], agg.at[didx.at[p, j]], ssems[b]).wait()

  def edge_loop(e_hbm):
    def idx_start(k, p):
      pltpu.async_copy(
          e_hbm.at[0, pl.ds(blk0 + k * _IB, _IB)], sidx.at[p], sisems[p])
      pltpu.async_copy(
          e_hbm.at[1, pl.ds(blk0 + k * _IB, _IB)], didx.at[p], disems[p])

    def idx_wait(k, p):
      pltpu.make_async_copy(
          e_hbm.at[0, pl.ds(blk0 + k * _IB, _IB)], sidx.at[p],
          sisems[p]).wait()
      pltpu.make_async_copy(
          e_hbm.at[1, pl.ds(blk0 + k * _IB, _IB)], didx.at[p],
          disems[p]).wait()

    idx_start(0, 0)

    def block(k, p):
      @pl.when(k + 1 < _NBLK)
      def _():
        idx_start(k + 1, 1 - p)

      idx_wait(k, p)
      gather_start(p, 0, 0)
      gather_start(p, 1, 1)
      for j in range(_IB):
        b = j % 2
        gather_wait(p, j, b)
        scatter_start(p, j, b)
        scatter_wait(p, j, b)
        if j + 2 < _IB:
          gather_start(p, j + 2, b)

    def two_blocks(i, carry):
      block(2 * i, 0)
      block(2 * i + 1, 1)
      return carry

    lax.fori_loop(0, _NBLK // 2, two_blocks, 0)
    block(_NBLK - 1, 0)

    @pl.when(s < _NTAIL)
    def _():
      pltpu.sync_copy(e_hbm.at[0, _NSUB * _CPT + s], sidx.at[1, 0])
      pltpu.sync_copy(e_hbm.at[1, _NSUB * _CPT + s], didx.at[1, 0])
      gather_start(1, 0, 0)
      gather_wait(1, 0, 0)
      scatter_start(1, 0, 0)
      scatter_wait(1, 0, 0)

  for i in range(_RPT // _RCH):
    pltpu.sync_copy(x_hbm.at[pl.ds(row0 + i * _RCH, _RCH)],
                    bufs.at[0, pl.ds(0, _RCH)])
    pltpu.sync_copy(bufs.at[0, pl.ds(0, _RCH)],
                    agg.at[pl.ds(row0 + i * _RCH, _RCH)])
  plsc.subcore_barrier()

  @pl.when(c == 0)
  def _():
    edge_loop(e0_hbm)

  @pl.when(c == 1)
  def _():
    edge_loop(e1_hbm)

  plsc.subcore_barrier()

  for i in range(_RPT // _RCH):
    pltpu.sync_copy(agg.at[pl.ds(row0 + i * _RCH, _RCH)],
                    bufs.at[0, pl.ds(0, _RCH)])
    pltpu.sync_copy(bufs.at[0, pl.ds(0, _RCH)],
                    out_hbm.at[pl.ds(c * _N + row0 + i * _RCH, _RCH)])


_sc_aggregate = functools.partial(
    pl.kernel,
    mesh=plsc.VectorSubcoreMesh(core_axis_name="c", subcore_axis_name="s"),
    compiler_params=pltpu.CompilerParams(use_tc_tiling_on_sc=False),
    out_type=jax.ShapeDtypeStruct((2 * _N, _D), jnp.float32),
    scratch_types=[
        pltpu.VMEM((2, _IB, _CH), jnp.int32),
        pltpu.VMEM((2, _IB, _CH), jnp.int32),
        pltpu.VMEM((2, _CH, _D), jnp.float32),
        pltpu.VMEM_SHARED((_N, _D), jnp.float32),
        [pltpu.SemaphoreType.DMA] * 2,
        [pltpu.SemaphoreType.DMA] * 2,
        [pltpu.SemaphoreType.DMA] * 2,
        [pltpu.SemaphoreType.DMA] * 2,
    ],
)(_sc_body)


def _bn_relu(t, g, b):
  mu = jnp.mean(t, axis=0, keepdims=True)
  var = jnp.mean((t - mu) * (t - mu), axis=0, keepdims=True)
  return jnp.maximum((t - mu) * lax.rsqrt(var + _EPS) * g + b, 0.0)


def _mlp(h, w1t, w2t, g1, b1, g2, b2):
  t = jnp.dot(h, w1t, preferred_element_type=jnp.float32)
  t = _bn_relu(t, g1, b1)
  t = jnp.dot(t, w2t, preferred_element_type=jnp.float32)
  return _bn_relu(t, g2, b2)


def _tc_body(h2, w1t0, w2t0, w1t1, w2t1,
             g10, b10, g20, b20, g11, b11, g21, b21, out):
  o0 = _mlp(h2[0:_N, :], w1t0[...], w2t0[...],
            g10[...], b10[...], g20[...], b20[...])
  o1 = _mlp(h2[_N:2 * _N, :], w1t1[...], w2t1[...],
            g11[...], b11[...], g21[...], b21[...])
  out[...] = o0 + o1


_tc_mlp = pl.pallas_call(
    _tc_body,
    out_shape=jax.ShapeDtypeStruct((_N, _D), jnp.float32),
)


@jax.jit
def kernel(x, edge_index_rel0, edge_index_rel1,
           W1_rel0, W2_rel0, g1_rel0, b1_rel0, g2_rel0, b2_rel0,
           W1_rel1, W2_rel1, g1_rel1, b1_rel1, g2_rel1, b2_rel1):
  e0 = edge_index_rel0.reshape(2, _NCHUNK, _CH)
  e1 = edge_index_rel1.reshape(2, _NCHUNK, _CH)
  h2 = _sc_aggregate(x, e0, e1)
  vec = lambda v: v.reshape(1, _D)
  return _tc_mlp(h2, W1_rel0.T, W2_rel0.T, W1_rel1.T, W2_rel1.T,
                 vec(g1_rel0), vec(b1_rel0), vec(g2_rel0), vec(b2_rel0),
                 vec(g1_rel1), vec(b1_rel1), vec(g2_rel1), vec(b2_rel1))
